# baseline (speedup 1.0000x reference)
import jax
import jax.numpy as jnp
from jax import lax
from jax.experimental import pallas as pl
from jax.experimental.pallas import tpu as pltpu
from jax.experimental.pallas import tpu_sc as plsc

NCORES = 2
NSUB = 16
NWORK = NCORES * NSUB
L = 16

NP = 8
NB = 1250
NBB = NB * NB
MA = 30000
NC = 640000
ND = 200000

TABLE_R = 393216
TABLE = NWORK * TABLE_R
DUMPCELL = TABLE - 8

DISP_PAD = 200704

W = 20480
NCP = NWORK * W
CH = 2048
NCHUNK = W // CH
NR = CH // 128
CROWS = NR + 1
CCAP = CROWS * 128

ACC = 200192
DUMP = ND
SL = ACC // NSUB
NG = 4
NPER = NSUB // NG
ZSL = NG * ACC // NSUB


def _mesh():
    return plsc.VectorSubcoreMesh(
        core_axis_name="c", subcore_axis_name="s",
        num_cores=NCORES, num_subcores=NSUB)


_SC_PARAMS = pltpu.CompilerParams(
    needs_layout_passes=False, use_tc_tiling_on_sc=False)


DPS = DISP_PAD // NSUB


def _cellmix(cell):
    return lax.shift_right_logical(cell * jnp.int32(-1640531527), 13) & 0x3FFFF


def _phase2_body(cx_hbm, cy_hbm, cz_hbm, bco_hbm, ia_hbm, ib_hbm,
                 p0_hbm, p1_hbm, p2_hbm, dflat_hbm, out_hbm, tbl_hbm,
                 bco_v, ia_v, ib_v, p0_v, p1_v, p2_v,
                 c1_i, c2_i, pc1_i, pc2_i, pos1_v, pos2_v,
                 crow0_v, crow3_v, csi1_v, csi2_v,
                 cp0_v, cp1_v, cp2_v, cib_v,
                 cx0x_v, cx0y_v, cx0z_v, cx3x_v, cx3y_v, cx3z_v, csval_v,
                 crow0_i, crow3_i, csi1_i, csi2_i, csval_i,
                 tbuf_v, obuf_v, acc_sh, sem, seml):
    core = lax.axis_index("c")
    sub = lax.axis_index("s")
    wid = core * NSUB + sub
    grp = sub // NPER
    parity = sub % NPER
    gbase = grp * ACC
    iota = lax.iota(jnp.int32, L)

    def zb(i, _):
        tbuf_v[pl.ds(i * L, L)] = jnp.zeros((L,), jnp.float32)
        return 0
    lax.fori_loop(0, 2048 // L, zb, 0)
    zbase = sub * ZSL
    nz = ZSL // 2048
    zt = ZSL - nz * 2048
    def za(i, _):
        pltpu.sync_copy(tbuf_v, acc_sh.at[pl.ds(zbase + i * 2048, 2048)])
        return 0
    lax.fori_loop(0, nz, za, 0)
    if zt:
        pltpu.sync_copy(tbuf_v.at[pl.ds(0, zt)],
                        acc_sh.at[pl.ds(zbase + nz * 2048, zt)])

    def zi(i, _):
        crow0_v[pl.ds(i * L, L)] = jnp.zeros((L,), jnp.int32)
        crow3_v[pl.ds(i * L, L)] = jnp.zeros((L,), jnp.int32)
        csi1_v[pl.ds(i * L, L)] = jnp.full((L,), gbase + DUMP, jnp.int32)
        csi2_v[pl.ds(i * L, L)] = jnp.full((L,), gbase + DUMP, jnp.int32)
        return 0
    lax.fori_loop(0, CCAP // L, zi, 0)

    pltpu.sync_copy(bco_hbm, bco_v)

    cbase = core * TABLE
    dumpcell = cbase + TABLE - 8
    p1base = sub * DPS

    def p1grp(g, _):
        pltpu.sync_copy(dflat_hbm.at[pl.ds(p1base + g * 2048, 2048)], ia_v)
        def p1r(r, _):
            for j in range(8):
                o = r * 128 + j * L
                cell = ia_v[pl.ds(o, L)]
                posn = p1base + g * 2048 + o + iota
                valid = posn < ND
                tcell = jnp.where(valid, cbase + cell, dumpcell)
                c1_i[r, pl.ds(j * L, L)] = tcell
                c2_i[r, pl.ds(j * L, L)] = posn ^ _cellmix(tcell)
            return 0
        lax.fori_loop(0, 16, p1r, 0)
        def p1s(r, _):
            pltpu.sync_copy(c2_i.at[r], tbl_hbm.at[c1_i.at[r]])
            return 0
        lax.fori_loop(0, 16, p1s, 0)
        return 0
    lax.fori_loop(0, DPS // 2048, p1grp, 0)
    tb = p1base + (DPS // 2048) * 2048
    pltpu.sync_copy(dflat_hbm.at[pl.ds(tb, 256)], ia_v.at[pl.ds(0, 256)])
    for r in range(2):
        for j in range(8):
            o = r * 128 + j * L
            cell = ia_v[pl.ds(o, L)]
            posn = tb + o + iota
            valid = posn < ND
            tcell = jnp.where(valid, cbase + cell, dumpcell)
            c1_i[r, pl.ds(j * L, L)] = tcell
            c2_i[r, pl.ds(j * L, L)] = posn ^ _cellmix(tcell)
    for r in range(2):
        pltpu.sync_copy(c2_i.at[r], tbl_hbm.at[c1_i.at[r]])

    plsc.subcore_barrier()

    def chunk(ch, _):
        base = wid * W + ch * CH
        dsb = pl.ds(base, CH)
        lds = [pltpu.async_copy(ia_hbm.at[dsb], ia_v, seml),
               pltpu.async_copy(ib_hbm.at[dsb], ib_v, seml),
               pltpu.async_copy(p0_hbm.at[dsb], p0_v, seml),
               pltpu.async_copy(p1_hbm.at[dsb], p1_v, seml),
               pltpu.async_copy(p2_hbm.at[dsb], p2_v, seml)]
        for c in lds:
            c.wait()

        def l1(r, _):
            for k in range(8):
                o = r * 128 + k * L
                ia = ia_v[pl.ds(o, L)]
                rr3 = ia & 2047
                rr0 = lax.shift_right_logical(ia, 11) & 2047
                pp0 = lax.shift_right_logical(ia, 22) & 15
                pb = cbase + pp0 * NBB
                c1_i[r, pl.ds(k * L, L)] = pb + rr0 * NB + rr3
                c2_i[r, pl.ds(k * L, L)] = pb + rr3 * NB + rr0
            return 0
        lax.fori_loop(0, NR, l1, 0)

        cps = []
        for k in range(NR):
            dsk = pl.ds(k * 128, 128)
            cps.append(pltpu.async_copy(
                tbl_hbm.at[c1_i.at[k]], pos1_v.at[dsk], sem))
            cps.append(pltpu.async_copy(
                tbl_hbm.at[c2_i.at[k]], pos2_v.at[dsk], sem))
        for c in cps:
            c.wait()

        def l1b(r, _):
            for k in range(8):
                o = r * 128 + k * L
                dsk = pl.ds(k * L, L)
                p1r = (pos1_v[pl.ds(o, L)] ^ _cellmix(c1_i[r, dsk])) & 0x3FFFF
                p2r = (pos2_v[pl.ds(o, L)] ^ _cellmix(c2_i[r, dsk])) & 0x3FFFF
                pc1_i[r, dsk] = jnp.where(p1r < ND, p1r, p1r - 62144)
                pc2_i[r, dsk] = jnp.where(p2r < ND, p2r, p2r - 62144)
            return 0
        lax.fori_loop(0, NR, l1b, 0)

        cps = []
        for k in range(NR):
            dsk = pl.ds(k * 128, 128)
            cps.append(pltpu.async_copy(
                dflat_hbm.at[pc1_i.at[k]], pos1_v.at[dsk], sem))
            cps.append(pltpu.async_copy(
                dflat_hbm.at[pc2_i.at[k]], pos2_v.at[dsk], sem))
        for c in cps:
            c.wait()

        def cp(r, cnt):
            for k in range(8):
                o = r * 128 + k * L
                dfl1 = pos1_v[pl.ds(o, L)]
                dfl2 = pos2_v[pl.ds(o, L)]
                pos1 = pc1_i[r, pl.ds(k * L, L)]
                pos2 = pc2_i[r, pl.ds(k * L, L)]
                cc1 = c1_i[r, pl.ds(k * L, L)]
                cc2 = c2_i[r, pl.ds(k * L, L)]
                ia = ia_v[pl.ds(o, L)]
                ib = ib_v[pl.ds(o, L)]
                rr3 = ia & 2047
                rr0 = lax.shift_right_logical(ia, 11) & 2047
                pp0 = lax.shift_right_logical(ia, 22) & 15
                pp3 = lax.shift_right_logical(ia, 26) & 15
                gid = base + o + iota
                real = gid < NC
                v1 = (dfl1 + cbase == cc1) & real
                v2 = (dfl2 + cbase == cc2) & (rr0 != rr3) & real
                live = v1 | v2
                off0 = plsc.load_gather(bco_v, [pp0 * NB + rr0])
                off3 = plsc.load_gather(bco_v, [pp3 * NB + rr3])
                row0 = pp0 * MA + off0 + (ib & 31)
                row3 = pp3 * MA + off3 + (lax.shift_right_logical(ib, 5) & 31)
                si1 = gbase + jnp.where(v1, pos1, DUMP)
                si2 = gbase + jnp.where(v2, pos2, DUMP)
                dc = pl.ds(cnt, L)
                plsc.store_compressed(crow0_v.at[dc], row0, mask=live)
                plsc.store_compressed(crow3_v.at[dc], row3, mask=live)
                plsc.store_compressed(csi1_v.at[dc], si1, mask=live)
                plsc.store_compressed(csi2_v.at[dc], si2, mask=live)
                plsc.store_compressed(cp0_v.at[dc], p0_v[pl.ds(o, L)],
                                      mask=live)
                plsc.store_compressed(cp1_v.at[dc], p1_v[pl.ds(o, L)],
                                      mask=live)
                plsc.store_compressed(cp2_v.at[dc], p2_v[pl.ds(o, L)],
                                      mask=live)
                plsc.store_compressed(cib_v.at[dc], ib, mask=live)
                cnt = cnt + jnp.max(plsc.all_reduce_population_count(live))
            return cnt
        cnt = lax.fori_loop(0, NR, cp, jnp.int32(0))
        nb = (cnt + 127) // 128

        def cg(r, _):
            dsr = pl.ds(r * 128, 128)
            g = [pltpu.async_copy(cx_hbm.at[crow0_i.at[r]],
                                  cx0x_v.at[dsr], sem),
                 pltpu.async_copy(cy_hbm.at[crow0_i.at[r]],
                                  cx0y_v.at[dsr], sem),
                 pltpu.async_copy(cz_hbm.at[crow0_i.at[r]],
                                  cx0z_v.at[dsr], sem),
                 pltpu.async_copy(cx_hbm.at[crow3_i.at[r]],
                                  cx3x_v.at[dsr], sem),
                 pltpu.async_copy(cy_hbm.at[crow3_i.at[r]],
                                  cx3y_v.at[dsr], sem),
                 pltpu.async_copy(cz_hbm.at[crow3_i.at[r]],
                                  cx3z_v.at[dsr], sem)]
            for cc in g:
                cc.wait()
            return 0

        def st(r, _):
            for k in range(8):
                o = r * 128 + k * L
                crow0_i[r, pl.ds(k * L, L)] = crow0_v[pl.ds(o, L)]
                crow3_i[r, pl.ds(k * L, L)] = crow3_v[pl.ds(o, L)]
            return 0
        lax.fori_loop(0, nb, st, 0)
        lax.fori_loop(0, nb, cg, 0)

        def l2(r, _):
            for k in range(8):
                o = r * 128 + k * L
                dx = cx0x_v[pl.ds(o, L)] - cx3x_v[pl.ds(o, L)]
                dy = cx0y_v[pl.ds(o, L)] - cx3y_v[pl.ds(o, L)]
                dz = cx0z_v[pl.ds(o, L)] - cx3z_v[pl.ds(o, L)]
                d2 = dx * dx + dy * dy + dz * dz + 1e-12
                bits = lax.bitcast_convert_type(d2, jnp.int32)
                yb = jnp.int32(0x5F3759DF) - lax.shift_right_arithmetic(bits, 1)
                y = lax.bitcast_convert_type(yb, jnp.float32)
                y = y * (1.5 - 0.5 * d2 * y * y)
                y = y * (1.5 - 0.5 * d2 * y * y)
                y = y * (1.5 - 0.5 * d2 * y * y)
                d = d2 * y
                pp0 = cp0_v[pl.ds(o, L)]
                pp1 = cp1_v[pl.ds(o, L)]
                pp2 = cp2_v[pl.ds(o, L)]
                fnv = lax.shift_right_logical(cib_v[pl.ds(o, L)], 10) & 1
                t = (d - 5.0 * pp0) / (pp1 + 0.5)
                s0 = t * t
                lb = 2.0 * pp0
                ub = lb + 4.0 * pp2 + 1.0
                e1 = jnp.maximum(lb - d, 0.0)
                e2 = jnp.maximum(d - ub, 0.0)
                s1 = e1 * e1 + e2 * e2
                csval_v[pl.ds(o, L)] = jnp.where(fnv == 0, s0, s1)
            return 0
        lax.fori_loop(0, nb, l2, 0)

        for j in range(8):
            csval_v[pl.ds(cnt + j * L, L)] = jnp.zeros((L,), jnp.float32)

        def st2(r, _):
            for k in range(8):
                o = r * 128 + k * L
                csi1_i[r, pl.ds(k * L, L)] = csi1_v[pl.ds(o, L)]
                csi2_i[r, pl.ds(k * L, L)] = csi2_v[pl.ds(o, L)]
                csval_i[r, pl.ds(k * L, L)] = csval_v[pl.ds(o, L)]
            return 0
        lax.fori_loop(0, nb, st2, 0)

        def sca(r, _):
            pltpu.sync_copy(csval_i.at[r], acc_sh.at[csi1_i.at[r]], add=True)
            pltpu.sync_copy(csval_i.at[r], acc_sh.at[csi2_i.at[r]], add=True)
            return 0
        for p in range(NPER):
            plsc.subcore_barrier()
            @pl.when(parity == p)
            def _():
                lax.fori_loop(0, nb, sca, 0)
        return 0

    lax.fori_loop(0, NCHUNK, chunk, 0)

    plsc.subcore_barrier()

    obase = sub * SL
    hbase = core * ACC + obase
    nblk = SL // 2048
    tail = SL - nblk * 2048

    def red_block(off, size):
        def zc(i, _):
            obuf_v[pl.ds(i * L, L)] = jnp.zeros((L,), jnp.float32)
            return 0
        lax.fori_loop(0, size // L, zc, 0)
        def rg(g, _):
            pltpu.sync_copy(
                acc_sh.at[pl.ds(g * ACC + obase + off, size)],
                tbuf_v.at[pl.ds(0, size)])
            def av(i, _):
                obuf_v[pl.ds(i * L, L)] = (obuf_v[pl.ds(i * L, L)]
                                           + tbuf_v[pl.ds(i * L, L)])
                return 0
            lax.fori_loop(0, size // L, av, 0)
            return 0
        lax.fori_loop(0, NG, rg, 0)
        pltpu.sync_copy(obuf_v.at[pl.ds(0, size)],
                        out_hbm.at[pl.ds(hbase + off, size)])

    def wo(i, _):
        red_block(i * 2048, 2048)
        return 0
    lax.fori_loop(0, nblk, wo, 0)
    if tail:
        red_block(nblk * 2048, tail)


def _add_body(a_ref, o_ref):
    o_ref[...] = a_ref[0] + a_ref[1]


def kernel(coords, constraint_params, block_coord_offset, constraint_atoms,
           constraint_function_inds, block_pair_dispatch_indices):
    assert coords.shape == (NP, MA, 3)
    assert constraint_atoms.shape == (NC, 4, 3)
    assert block_pair_dispatch_indices.shape == (3, ND)
    assert block_coord_offset.shape == (NP, NB)

    cf = coords.reshape(NP * MA, 3)
    cx = cf[:, 0]
    cy = cf[:, 1]
    cz = cf[:, 2]
    bco = block_coord_offset.reshape(-1).astype(jnp.int32)

    pose0 = constraint_atoms[:, 0, 0]
    pose3 = constraint_atoms[:, 3, 0]
    r0 = constraint_atoms[:, 0, 1]
    a0 = constraint_atoms[:, 0, 2]
    r3 = constraint_atoms[:, 3, 1]
    a3 = constraint_atoms[:, 3, 2]
    fni = constraint_function_inds
    ia = r3 + (r0 << 11) + (pose0 << 22) + (pose3 << 26)
    ib = a0 + (a3 << 5) + (fni << 10)
    padc = NCP - NC
    pads = lambda x: jnp.pad(x, (0, padc))
    ia = pads(ia)
    ib = pads(ib)
    p0 = pads(constraint_params[:, 0])
    p1 = pads(constraint_params[:, 1])
    p2 = pads(constraint_params[:, 2])

    d0 = block_pair_dispatch_indices[0]
    d1 = block_pair_dispatch_indices[1]
    d2 = block_pair_dispatch_indices[2]
    dflat = d0 * NBB + d1 * NB + d2
    dflat_pad = jnp.pad(dflat, (0, DISP_PAD - ND))

    partials, _tbl = pl.kernel(
        _phase2_body,
        out_type=(jax.ShapeDtypeStruct((NCORES * ACC,), jnp.float32),
                  jax.ShapeDtypeStruct((NCORES * TABLE,), jnp.int32)),
        mesh=_mesh(),
        compiler_params=_SC_PARAMS,
        scratch_types=[
            pltpu.VMEM((NP * NB,), jnp.int32),
            pltpu.VMEM((CH,), jnp.int32),
            pltpu.VMEM((CH,), jnp.int32),
            pltpu.VMEM((CH,), jnp.float32),
            pltpu.VMEM((CH,), jnp.float32),
            pltpu.VMEM((CH,), jnp.float32),
            pltpu.VMEM((NR, 128), jnp.int32),
            pltpu.VMEM((NR, 128), jnp.int32),
            pltpu.VMEM((NR, 128), jnp.int32),
            pltpu.VMEM((NR, 128), jnp.int32),
            pltpu.VMEM((CH,), jnp.int32),
            pltpu.VMEM((CH,), jnp.int32),
            pltpu.VMEM((CCAP,), jnp.int32),
            pltpu.VMEM((CCAP,), jnp.int32),
            pltpu.VMEM((CCAP,), jnp.int32),
            pltpu.VMEM((CCAP,), jnp.int32),
            pltpu.VMEM((CCAP,), jnp.float32),
            pltpu.VMEM((CCAP,), jnp.float32),
            pltpu.VMEM((CCAP,), jnp.float32),
            pltpu.VMEM((CCAP,), jnp.int32),
            pltpu.VMEM((CCAP,), jnp.float32),
            pltpu.VMEM((CCAP,), jnp.float32),
            pltpu.VMEM((CCAP,), jnp.float32),
            pltpu.VMEM((CCAP,), jnp.float32),
            pltpu.VMEM((CCAP,), jnp.float32),
            pltpu.VMEM((CCAP,), jnp.float32),
            pltpu.VMEM((CCAP,), jnp.float32),
            pltpu.VMEM((CROWS, 128), jnp.int32),
            pltpu.VMEM((CROWS, 128), jnp.int32),
            pltpu.VMEM((CROWS, 128), jnp.int32),
            pltpu.VMEM((CROWS, 128), jnp.int32),
            pltpu.VMEM((CROWS, 128), jnp.float32),
            pltpu.VMEM((2048,), jnp.float32),
            pltpu.VMEM((2048,), jnp.float32),
            pltpu.VMEM_SHARED((NG * ACC,), jnp.float32),
            pltpu.SemaphoreType.DMA,
            pltpu.SemaphoreType.DMA,
        ],
    )(cx, cy, cz, bco, ia, ib, p0, p1, p2, dflat_pad)

    summed = pl.pallas_call(
        _add_body,
        out_shape=jax.ShapeDtypeStruct((ACC // 128, 128), jnp.float32),
    )(partials.reshape(NCORES, ACC // 128, 128))

    return summed.reshape(-1)[:ND]


# speedup vs baseline: 1.1691x; 1.1691x over previous
import jax
import jax.numpy as jnp
from jax import lax
from jax.experimental import pallas as pl
from jax.experimental.pallas import tpu as pltpu
from jax.experimental.pallas import tpu_sc as plsc

NCORES = 2
NSUB = 16
NWORK = NCORES * NSUB
L = 16

NP = 8
NB = 1250
NBB = NB * NB
MA = 30000
NC = 640000
ND = 200000

TABLE_R = 393216
TABLE = NWORK * TABLE_R
DUMPCELL = TABLE - 8

DISP_PAD = 200704

W = 20480
NCP = NWORK * W
CH = 2048
NCHUNK = W // CH
NR = CH // 128
CROWS = NR + 1
CCAP = CROWS * 128

ACC = 200192
DUMP = ND
SL = ACC // NSUB
NG = 4
NPER = NSUB // NG
ZSL = NG * ACC // NSUB


def _mesh():
    return plsc.VectorSubcoreMesh(
        core_axis_name="c", subcore_axis_name="s",
        num_cores=NCORES, num_subcores=NSUB)


_SC_PARAMS = pltpu.CompilerParams(
    needs_layout_passes=False, use_tc_tiling_on_sc=False)


DPW = DISP_PAD // NWORK


def _cellmix(cell):
    return lax.shift_right_logical(cell * jnp.int32(-1640531527), 13) & 0x3FFFF


def _phase1_body(disp_hbm, lookup_hbm, dchunk_v, tgt_v, val_v, sem):
    core = lax.axis_index("c")
    sub = lax.axis_index("s")
    wid = core * NSUB + sub
    iota = lax.iota(jnp.int32, L)
    base = wid * DPW
    pltpu.sync_copy(disp_hbm.at[pl.ds(base, DPW)], dchunk_v)

    def cmp_(r, _):
        for j in range(8):
            o = r * 128 + j * L
            cell = dchunk_v[pl.ds(o, L)]
            posn = base + o + iota
            valid = posn < ND
            tgt_v[r, pl.ds(j * L, L)] = jnp.where(valid, cell, DUMPCELL)
            val_v[r, pl.ds(j * L, L)] = posn ^ _cellmix(cell)
        return 0
    lax.fori_loop(0, DPW // 128, cmp_, 0)

    for g in range(7):
        cps = [pltpu.async_copy(
            val_v.at[g * 7 + r], lookup_hbm.at[tgt_v.at[g * 7 + r]], sem)
            for r in range(7)]
        for c in cps:
            c.wait()


def _phase2_body(cx_hbm, cy_hbm, cz_hbm, bco_hbm, ia_hbm, ib_hbm,
                 p0_hbm, p1_hbm, p2_hbm, tbl_hbm, dflat_hbm, out_hbm,
                 bco_v, ia_v, ib_v, p0_v, p1_v, p2_v,
                 c1_i, c2_i, pc1_i, pc2_i, pos1_v, pos2_v,
                 crow0_v, crow3_v, csi1_v, csi2_v,
                 cp0_v, cp1_v, cp2_v, cib_v,
                 cx0x_v, cx0y_v, cx0z_v, cx3x_v, cx3y_v, cx3z_v, csval_v,
                 crow0_i, crow3_i, csi1_i, csi2_i, csval_i,
                 tbuf_v, obuf_v, acc_sh, sem, seml):
    core = lax.axis_index("c")
    sub = lax.axis_index("s")
    wid = core * NSUB + sub
    grp = sub // NPER
    parity = sub % NPER
    gbase = grp * ACC
    iota = lax.iota(jnp.int32, L)

    def zb(i, _):
        tbuf_v[pl.ds(i * L, L)] = jnp.zeros((L,), jnp.float32)
        return 0
    lax.fori_loop(0, 2048 // L, zb, 0)
    zbase = sub * ZSL
    nz = ZSL // 2048
    zt = ZSL - nz * 2048
    def za(i, _):
        pltpu.sync_copy(tbuf_v, acc_sh.at[pl.ds(zbase + i * 2048, 2048)])
        return 0
    lax.fori_loop(0, nz, za, 0)
    if zt:
        pltpu.sync_copy(tbuf_v.at[pl.ds(0, zt)],
                        acc_sh.at[pl.ds(zbase + nz * 2048, zt)])

    def zi(i, _):
        crow0_v[pl.ds(i * L, L)] = jnp.zeros((L,), jnp.int32)
        crow3_v[pl.ds(i * L, L)] = jnp.zeros((L,), jnp.int32)
        csi1_v[pl.ds(i * L, L)] = jnp.full((L,), gbase + DUMP, jnp.int32)
        csi2_v[pl.ds(i * L, L)] = jnp.full((L,), gbase + DUMP, jnp.int32)
        return 0
    lax.fori_loop(0, CCAP // L, zi, 0)

    pltpu.sync_copy(bco_hbm, bco_v)
    plsc.subcore_barrier()

    def chunk(ch, _):
        base = wid * W + ch * CH
        dsb = pl.ds(base, CH)
        lds = [pltpu.async_copy(ia_hbm.at[dsb], ia_v, seml),
               pltpu.async_copy(ib_hbm.at[dsb], ib_v, seml),
               pltpu.async_copy(p0_hbm.at[dsb], p0_v, seml),
               pltpu.async_copy(p1_hbm.at[dsb], p1_v, seml),
               pltpu.async_copy(p2_hbm.at[dsb], p2_v, seml)]
        for c in lds:
            c.wait()

        def l1(r, _):
            for k in range(8):
                o = r * 128 + k * L
                ia = ia_v[pl.ds(o, L)]
                rr3 = ia & 2047
                rr0 = lax.shift_right_logical(ia, 11) & 2047
                pp0 = lax.shift_right_logical(ia, 22) & 15
                pb = pp0 * NBB
                c1_i[r, pl.ds(k * L, L)] = pb + rr0 * NB + rr3
                c2_i[r, pl.ds(k * L, L)] = pb + rr3 * NB + rr0
            return 0
        lax.fori_loop(0, NR, l1, 0)

        cps = []
        for k in range(NR):
            dsk = pl.ds(k * 128, 128)
            cps.append(pltpu.async_copy(
                tbl_hbm.at[c1_i.at[k]], pos1_v.at[dsk], sem))
            cps.append(pltpu.async_copy(
                tbl_hbm.at[c2_i.at[k]], pos2_v.at[dsk], sem))
        for c in cps:
            c.wait()

        def l1b(r, _):
            for k in range(8):
                o = r * 128 + k * L
                dsk = pl.ds(k * L, L)
                p1r = (pos1_v[pl.ds(o, L)] ^ _cellmix(c1_i[r, dsk])) & 0x3FFFF
                p2r = (pos2_v[pl.ds(o, L)] ^ _cellmix(c2_i[r, dsk])) & 0x3FFFF
                pc1_i[r, dsk] = jnp.where(p1r < ND, p1r, p1r - 62144)
                pc2_i[r, dsk] = jnp.where(p2r < ND, p2r, p2r - 62144)
            return 0
        lax.fori_loop(0, NR, l1b, 0)

        cps = []
        for k in range(NR):
            dsk = pl.ds(k * 128, 128)
            cps.append(pltpu.async_copy(
                dflat_hbm.at[pc1_i.at[k]], pos1_v.at[dsk], sem))
            cps.append(pltpu.async_copy(
                dflat_hbm.at[pc2_i.at[k]], pos2_v.at[dsk], sem))
        for c in cps:
            c.wait()

        def cp(r, cnt):
            for k in range(8):
                o = r * 128 + k * L
                dfl1 = pos1_v[pl.ds(o, L)]
                dfl2 = pos2_v[pl.ds(o, L)]
                pos1 = pc1_i[r, pl.ds(k * L, L)]
                pos2 = pc2_i[r, pl.ds(k * L, L)]
                cc1 = c1_i[r, pl.ds(k * L, L)]
                cc2 = c2_i[r, pl.ds(k * L, L)]
                ia = ia_v[pl.ds(o, L)]
                ib = ib_v[pl.ds(o, L)]
                rr3 = ia & 2047
                rr0 = lax.shift_right_logical(ia, 11) & 2047
                pp0 = lax.shift_right_logical(ia, 22) & 15
                pp3 = lax.shift_right_logical(ia, 26) & 15
                gid = base + o + iota
                real = gid < NC
                v1 = (dfl1 == cc1) & real
                v2 = (dfl2 == cc2) & (rr0 != rr3) & real
                live = v1 | v2
                off0 = plsc.load_gather(bco_v, [pp0 * NB + rr0])
                off3 = plsc.load_gather(bco_v, [pp3 * NB + rr3])
                row0 = pp0 * MA + off0 + (ib & 31)
                row3 = pp3 * MA + off3 + (lax.shift_right_logical(ib, 5) & 31)
                si1 = gbase + jnp.where(v1, pos1, DUMP)
                si2 = gbase + jnp.where(v2, pos2, DUMP)
                dc = pl.ds(cnt, L)
                plsc.store_compressed(crow0_v.at[dc], row0, mask=live)
                plsc.store_compressed(crow3_v.at[dc], row3, mask=live)
                plsc.store_compressed(csi1_v.at[dc], si1, mask=live)
                plsc.store_compressed(csi2_v.at[dc], si2, mask=live)
                plsc.store_compressed(cp0_v.at[dc], p0_v[pl.ds(o, L)],
                                      mask=live)
                plsc.store_compressed(cp1_v.at[dc], p1_v[pl.ds(o, L)],
                                      mask=live)
                plsc.store_compressed(cp2_v.at[dc], p2_v[pl.ds(o, L)],
                                      mask=live)
                plsc.store_compressed(cib_v.at[dc], ib, mask=live)
                cnt = cnt + jnp.max(plsc.all_reduce_population_count(live))
            return cnt
        cnt = lax.fori_loop(0, NR, cp, jnp.int32(0))
        nb = (cnt + 127) // 128

        def cg(r, _):
            dsr = pl.ds(r * 128, 128)
            g = [pltpu.async_copy(cx_hbm.at[crow0_i.at[r]],
                                  cx0x_v.at[dsr], sem),
                 pltpu.async_copy(cy_hbm.at[crow0_i.at[r]],
                                  cx0y_v.at[dsr], sem),
                 pltpu.async_copy(cz_hbm.at[crow0_i.at[r]],
                                  cx0z_v.at[dsr], sem),
                 pltpu.async_copy(cx_hbm.at[crow3_i.at[r]],
                                  cx3x_v.at[dsr], sem),
                 pltpu.async_copy(cy_hbm.at[crow3_i.at[r]],
                                  cx3y_v.at[dsr], sem),
                 pltpu.async_copy(cz_hbm.at[crow3_i.at[r]],
                                  cx3z_v.at[dsr], sem)]
            for cc in g:
                cc.wait()
            return 0

        def st(r, _):
            for k in range(8):
                o = r * 128 + k * L
                crow0_i[r, pl.ds(k * L, L)] = crow0_v[pl.ds(o, L)]
                crow3_i[r, pl.ds(k * L, L)] = crow3_v[pl.ds(o, L)]
            return 0
        lax.fori_loop(0, nb, st, 0)
        lax.fori_loop(0, nb, cg, 0)

        def l2(r, _):
            for k in range(8):
                o = r * 128 + k * L
                dx = cx0x_v[pl.ds(o, L)] - cx3x_v[pl.ds(o, L)]
                dy = cx0y_v[pl.ds(o, L)] - cx3y_v[pl.ds(o, L)]
                dz = cx0z_v[pl.ds(o, L)] - cx3z_v[pl.ds(o, L)]
                d2 = dx * dx + dy * dy + dz * dz + 1e-12
                bits = lax.bitcast_convert_type(d2, jnp.int32)
                yb = jnp.int32(0x5F3759DF) - lax.shift_right_arithmetic(bits, 1)
                y = lax.bitcast_convert_type(yb, jnp.float32)
                y = y * (1.5 - 0.5 * d2 * y * y)
                y = y * (1.5 - 0.5 * d2 * y * y)
                y = y * (1.5 - 0.5 * d2 * y * y)
                d = d2 * y
                pp0 = cp0_v[pl.ds(o, L)]
                pp1 = cp1_v[pl.ds(o, L)]
                pp2 = cp2_v[pl.ds(o, L)]
                fnv = lax.shift_right_logical(cib_v[pl.ds(o, L)], 10) & 1
                t = (d - 5.0 * pp0) / (pp1 + 0.5)
                s0 = t * t
                lb = 2.0 * pp0
                ub = lb + 4.0 * pp2 + 1.0
                e1 = jnp.maximum(lb - d, 0.0)
                e2 = jnp.maximum(d - ub, 0.0)
                s1 = e1 * e1 + e2 * e2
                csval_v[pl.ds(o, L)] = jnp.where(fnv == 0, s0, s1)
            return 0
        lax.fori_loop(0, nb, l2, 0)

        for j in range(8):
            csval_v[pl.ds(cnt + j * L, L)] = jnp.zeros((L,), jnp.float32)

        def st2(r, _):
            for k in range(8):
                o = r * 128 + k * L
                csi1_i[r, pl.ds(k * L, L)] = csi1_v[pl.ds(o, L)]
                csi2_i[r, pl.ds(k * L, L)] = csi2_v[pl.ds(o, L)]
                csval_i[r, pl.ds(k * L, L)] = csval_v[pl.ds(o, L)]
            return 0
        lax.fori_loop(0, nb, st2, 0)

        def sca(r, _):
            pltpu.sync_copy(csval_i.at[r], acc_sh.at[csi1_i.at[r]], add=True)
            pltpu.sync_copy(csval_i.at[r], acc_sh.at[csi2_i.at[r]], add=True)
            return 0
        for p in range(NPER):
            plsc.subcore_barrier()
            @pl.when(parity == p)
            def _():
                lax.fori_loop(0, nb, sca, 0)
        return 0

    lax.fori_loop(0, NCHUNK, chunk, 0)

    plsc.subcore_barrier()

    obase = sub * SL
    hbase = core * ACC + obase
    nblk = SL // 2048
    tail = SL - nblk * 2048

    def red_block(off, size):
        def zc(i, _):
            obuf_v[pl.ds(i * L, L)] = jnp.zeros((L,), jnp.float32)
            return 0
        lax.fori_loop(0, size // L, zc, 0)
        def rg(g, _):
            pltpu.sync_copy(
                acc_sh.at[pl.ds(g * ACC + obase + off, size)],
                tbuf_v.at[pl.ds(0, size)])
            def av(i, _):
                obuf_v[pl.ds(i * L, L)] = (obuf_v[pl.ds(i * L, L)]
                                           + tbuf_v[pl.ds(i * L, L)])
                return 0
            lax.fori_loop(0, size // L, av, 0)
            return 0
        lax.fori_loop(0, NG, rg, 0)
        pltpu.sync_copy(obuf_v.at[pl.ds(0, size)],
                        out_hbm.at[pl.ds(hbase + off, size)])

    def wo(i, _):
        red_block(i * 2048, 2048)
        return 0
    lax.fori_loop(0, nblk, wo, 0)
    if tail:
        red_block(nblk * 2048, tail)


def _add_body(a_ref, o_ref):
    o_ref[...] = a_ref[0] + a_ref[1]


def kernel(coords, constraint_params, block_coord_offset, constraint_atoms,
           constraint_function_inds, block_pair_dispatch_indices):
    assert coords.shape == (NP, MA, 3)
    assert constraint_atoms.shape == (NC, 4, 3)
    assert block_pair_dispatch_indices.shape == (3, ND)
    assert block_coord_offset.shape == (NP, NB)

    cf = coords.reshape(NP * MA, 3)
    cx = cf[:, 0]
    cy = cf[:, 1]
    cz = cf[:, 2]
    bco = block_coord_offset.reshape(-1).astype(jnp.int32)

    pose0 = constraint_atoms[:, 0, 0]
    pose3 = constraint_atoms[:, 3, 0]
    r0 = constraint_atoms[:, 0, 1]
    a0 = constraint_atoms[:, 0, 2]
    r3 = constraint_atoms[:, 3, 1]
    a3 = constraint_atoms[:, 3, 2]
    fni = constraint_function_inds
    ia = r3 + (r0 << 11) + (pose0 << 22) + (pose3 << 26)
    ib = a0 + (a3 << 5) + (fni << 10)
    padc = NCP - NC
    pads = lambda x: jnp.pad(x, (0, padc))
    ia = pads(ia)
    ib = pads(ib)
    p0 = pads(constraint_params[:, 0])
    p1 = pads(constraint_params[:, 1])
    p2 = pads(constraint_params[:, 2])

    d0 = block_pair_dispatch_indices[0]
    d1 = block_pair_dispatch_indices[1]
    d2 = block_pair_dispatch_indices[2]
    dflat = d0 * NBB + d1 * NB + d2
    dflat_pad = jnp.pad(dflat, (0, DISP_PAD - ND))

    lookup = pl.kernel(
        _phase1_body,
        out_type=jax.ShapeDtypeStruct((TABLE,), jnp.int32),
        mesh=_mesh(),
        compiler_params=_SC_PARAMS,
        scratch_types=[
            pltpu.VMEM((DPW,), jnp.int32),
            pltpu.VMEM((DPW // 128, 128), jnp.int32),
            pltpu.VMEM((DPW // 128, 128), jnp.int32),
            pltpu.SemaphoreType.DMA,
        ],
    )(dflat_pad)

    partials = pl.kernel(
        _phase2_body,
        out_type=jax.ShapeDtypeStruct((NCORES * ACC,), jnp.float32),
        mesh=_mesh(),
        compiler_params=_SC_PARAMS,
        scratch_types=[
            pltpu.VMEM((NP * NB,), jnp.int32),
            pltpu.VMEM((CH,), jnp.int32),
            pltpu.VMEM((CH,), jnp.int32),
            pltpu.VMEM((CH,), jnp.float32),
            pltpu.VMEM((CH,), jnp.float32),
            pltpu.VMEM((CH,), jnp.float32),
            pltpu.VMEM((NR, 128), jnp.int32),
            pltpu.VMEM((NR, 128), jnp.int32),
            pltpu.VMEM((NR, 128), jnp.int32),
            pltpu.VMEM((NR, 128), jnp.int32),
            pltpu.VMEM((CH,), jnp.int32),
            pltpu.VMEM((CH,), jnp.int32),
            pltpu.VMEM((CCAP,), jnp.int32),
            pltpu.VMEM((CCAP,), jnp.int32),
            pltpu.VMEM((CCAP,), jnp.int32),
            pltpu.VMEM((CCAP,), jnp.int32),
            pltpu.VMEM((CCAP,), jnp.float32),
            pltpu.VMEM((CCAP,), jnp.float32),
            pltpu.VMEM((CCAP,), jnp.float32),
            pltpu.VMEM((CCAP,), jnp.int32),
            pltpu.VMEM((CCAP,), jnp.float32),
            pltpu.VMEM((CCAP,), jnp.float32),
            pltpu.VMEM((CCAP,), jnp.float32),
            pltpu.VMEM((CCAP,), jnp.float32),
            pltpu.VMEM((CCAP,), jnp.float32),
            pltpu.VMEM((CCAP,), jnp.float32),
            pltpu.VMEM((CCAP,), jnp.float32),
            pltpu.VMEM((CROWS, 128), jnp.int32),
            pltpu.VMEM((CROWS, 128), jnp.int32),
            pltpu.VMEM((CROWS, 128), jnp.int32),
            pltpu.VMEM((CROWS, 128), jnp.int32),
            pltpu.VMEM((CROWS, 128), jnp.float32),
            pltpu.VMEM((2048,), jnp.float32),
            pltpu.VMEM((2048,), jnp.float32),
            pltpu.VMEM_SHARED((NG * ACC,), jnp.float32),
            pltpu.SemaphoreType.DMA,
            pltpu.SemaphoreType.DMA,
        ],
    )(cx, cy, cz, bco, ia, ib, p0, p1, p2, lookup, dflat_pad)

    summed = pl.pallas_call(
        _add_body,
        out_shape=jax.ShapeDtypeStruct((ACC // 128, 128), jnp.float32),
    )(partials.reshape(NCORES, ACC // 128, 128))

    return summed.reshape(-1)[:ND]


# speedup vs baseline: 1.1694x; 1.0002x over previous
import jax
import jax.numpy as jnp
from jax import lax
from jax.experimental import pallas as pl
from jax.experimental.pallas import tpu as pltpu
from jax.experimental.pallas import tpu_sc as plsc

NCORES = 2
NSUB = 16
NWORK = NCORES * NSUB
L = 16

NP = 8
NB = 1250
NBB = NB * NB
MA = 30000
NC = 640000
ND = 200000

TABLE_R = 393216
TABLE = NWORK * TABLE_R
DUMPCELL = TABLE - 8

DISP_PAD = 200704

W = 20480
NCP = NWORK * W
CH = 2048
NCHUNK = W // CH
NR = CH // 128
CROWS = NR + 1
CCAP = CROWS * 128

ACC = 200192
DUMP = ND
SL = ACC // NSUB
NG = 4
NPER = NSUB // NG
ZSL = NG * ACC // NSUB


def _mesh():
    return plsc.VectorSubcoreMesh(
        core_axis_name="c", subcore_axis_name="s",
        num_cores=NCORES, num_subcores=NSUB)


_SC_PARAMS = pltpu.CompilerParams(
    needs_layout_passes=False, use_tc_tiling_on_sc=False)


DPW = DISP_PAD // NWORK


def _cellmix(cell):
    return lax.shift_right_logical(cell * jnp.int32(-1640531527), 13) & 0x3FFFF


def _phase1_body(disp_hbm, lookup_hbm, dchunk_v, tgt_v, val_v, sem):
    core = lax.axis_index("c")
    sub = lax.axis_index("s")
    wid = core * NSUB + sub
    iota = lax.iota(jnp.int32, L)
    base = wid * DPW
    pltpu.sync_copy(disp_hbm.at[pl.ds(base, DPW)], dchunk_v)

    def cmp_(r, _):
        for j in range(8):
            o = r * 128 + j * L
            cell = dchunk_v[pl.ds(o, L)]
            posn = base + o + iota
            valid = posn < ND
            tgt_v[r, pl.ds(j * L, L)] = jnp.where(valid, cell, DUMPCELL)
            val_v[r, pl.ds(j * L, L)] = posn ^ _cellmix(cell)
        return 0
    lax.fori_loop(0, DPW // 128, cmp_, 0)

    for g in range(7):
        cps = [pltpu.async_copy(
            val_v.at[g * 7 + r], lookup_hbm.at[tgt_v.at[g * 7 + r]], sem)
            for r in range(7)]
        for c in cps:
            c.wait()


def _phase2_body(cx_hbm, cy_hbm, cz_hbm, bco_hbm, ia_hbm, ib_hbm,
                 p0_hbm, p1_hbm, p2_hbm, tbl_hbm, dflat_hbm, out_hbm,
                 bco_v, ia_v, ib_v, p0_v, p1_v, p2_v,
                 c1_i, c2_i, pc1_i, pc2_i, pos1_v, pos2_v,
                 crow0_v, crow3_v, csi1_v, csi2_v,
                 cp0_v, cp1_v, cp2_v, cib_v,
                 cx0x_v, cx0y_v, cx0z_v, cx3x_v, cx3y_v, cx3z_v, csval_v,
                 crow0_i, crow3_i, csi1_i, csi2_i, csval_i,
                 tbuf_v, obuf_v, acc_sh, sem, seml):
    core = lax.axis_index("c")
    sub = lax.axis_index("s")
    wid = core * NSUB + sub
    grp = sub // NPER
    parity = sub % NPER
    gbase = grp * ACC
    iota = lax.iota(jnp.int32, L)

    def zb(i, _):
        tbuf_v[pl.ds(i * L, L)] = jnp.zeros((L,), jnp.float32)
        return 0
    lax.fori_loop(0, 2048 // L, zb, 0)
    zbase = sub * ZSL
    nz = ZSL // 2048
    zt = ZSL - nz * 2048
    def za(i, _):
        pltpu.sync_copy(tbuf_v, acc_sh.at[pl.ds(zbase + i * 2048, 2048)])
        return 0
    lax.fori_loop(0, nz, za, 0)
    if zt:
        pltpu.sync_copy(tbuf_v.at[pl.ds(0, zt)],
                        acc_sh.at[pl.ds(zbase + nz * 2048, zt)])

    def zi(i, _):
        crow0_v[pl.ds(i * L, L)] = jnp.zeros((L,), jnp.int32)
        crow3_v[pl.ds(i * L, L)] = jnp.zeros((L,), jnp.int32)
        csi1_v[pl.ds(i * L, L)] = jnp.full((L,), gbase + DUMP, jnp.int32)
        csi2_v[pl.ds(i * L, L)] = jnp.full((L,), gbase + DUMP, jnp.int32)
        return 0
    lax.fori_loop(0, CCAP // L, zi, 0)

    pltpu.sync_copy(bco_hbm, bco_v)
    plsc.subcore_barrier()

    def chunk(ch, _):
        base = wid * W + ch * CH
        dsb = pl.ds(base, CH)
        lds = [pltpu.async_copy(ia_hbm.at[dsb], ia_v, seml),
               pltpu.async_copy(ib_hbm.at[dsb], ib_v, seml),
               pltpu.async_copy(p0_hbm.at[dsb], p0_v, seml),
               pltpu.async_copy(p1_hbm.at[dsb], p1_v, seml),
               pltpu.async_copy(p2_hbm.at[dsb], p2_v, seml)]
        for c in lds:
            c.wait()

        def l1(r, _):
            for k in range(8):
                o = r * 128 + k * L
                ia = ia_v[pl.ds(o, L)]
                rr3 = ia & 2047
                rr0 = lax.shift_right_logical(ia, 11) & 2047
                pp0 = lax.shift_right_logical(ia, 22) & 15
                pb = pp0 * NBB
                c1_i[pl.ds(o, L)] = pb + rr0 * NB + rr3
                c2_i[pl.ds(o, L)] = pb + rr3 * NB + rr0
            return 0
        lax.fori_loop(0, NR, l1, 0)

        cps = [pltpu.async_copy(tbl_hbm.at[c1_i], pos1_v, sem),
               pltpu.async_copy(tbl_hbm.at[c2_i], pos2_v, sem)]
        for c in cps:
            c.wait()

        def l1b(r, _):
            for k in range(8):
                o = r * 128 + k * L
                dso = pl.ds(o, L)
                p1r = (pos1_v[dso] ^ _cellmix(c1_i[dso])) & 0x3FFFF
                p2r = (pos2_v[dso] ^ _cellmix(c2_i[dso])) & 0x3FFFF
                pc1_i[dso] = jnp.where(p1r < ND, p1r, p1r - 62144)
                pc2_i[dso] = jnp.where(p2r < ND, p2r, p2r - 62144)
            return 0
        lax.fori_loop(0, NR, l1b, 0)

        cps = [pltpu.async_copy(dflat_hbm.at[pc1_i], pos1_v, sem),
               pltpu.async_copy(dflat_hbm.at[pc2_i], pos2_v, sem)]
        for c in cps:
            c.wait()

        def cp(r, cnt):
            for k in range(8):
                o = r * 128 + k * L
                dfl1 = pos1_v[pl.ds(o, L)]
                dfl2 = pos2_v[pl.ds(o, L)]
                pos1 = pc1_i[pl.ds(o, L)]
                pos2 = pc2_i[pl.ds(o, L)]
                cc1 = c1_i[pl.ds(o, L)]
                cc2 = c2_i[pl.ds(o, L)]
                ia = ia_v[pl.ds(o, L)]
                ib = ib_v[pl.ds(o, L)]
                rr3 = ia & 2047
                rr0 = lax.shift_right_logical(ia, 11) & 2047
                pp0 = lax.shift_right_logical(ia, 22) & 15
                pp3 = lax.shift_right_logical(ia, 26) & 15
                gid = base + o + iota
                real = gid < NC
                v1 = (dfl1 == cc1) & real
                v2 = (dfl2 == cc2) & (rr0 != rr3) & real
                live = v1 | v2
                off0 = plsc.load_gather(bco_v, [pp0 * NB + rr0])
                off3 = plsc.load_gather(bco_v, [pp3 * NB + rr3])
                row0 = pp0 * MA + off0 + (ib & 31)
                row3 = pp3 * MA + off3 + (lax.shift_right_logical(ib, 5) & 31)
                si1 = gbase + jnp.where(v1, pos1, DUMP)
                si2 = gbase + jnp.where(v2, pos2, DUMP)
                dc = pl.ds(cnt, L)
                plsc.store_compressed(crow0_v.at[dc], row0, mask=live)
                plsc.store_compressed(crow3_v.at[dc], row3, mask=live)
                plsc.store_compressed(csi1_v.at[dc], si1, mask=live)
                plsc.store_compressed(csi2_v.at[dc], si2, mask=live)
                plsc.store_compressed(cp0_v.at[dc], p0_v[pl.ds(o, L)],
                                      mask=live)
                plsc.store_compressed(cp1_v.at[dc], p1_v[pl.ds(o, L)],
                                      mask=live)
                plsc.store_compressed(cp2_v.at[dc], p2_v[pl.ds(o, L)],
                                      mask=live)
                plsc.store_compressed(cib_v.at[dc], ib, mask=live)
                cnt = cnt + jnp.max(plsc.all_reduce_population_count(live))
            return cnt
        cnt = lax.fori_loop(0, NR, cp, jnp.int32(0))
        nb = (cnt + 127) // 128

        def cg(r, _):
            dsr = pl.ds(r * 128, 128)
            g = [pltpu.async_copy(cx_hbm.at[crow0_i.at[r]],
                                  cx0x_v.at[dsr], sem),
                 pltpu.async_copy(cy_hbm.at[crow0_i.at[r]],
                                  cx0y_v.at[dsr], sem),
                 pltpu.async_copy(cz_hbm.at[crow0_i.at[r]],
                                  cx0z_v.at[dsr], sem),
                 pltpu.async_copy(cx_hbm.at[crow3_i.at[r]],
                                  cx3x_v.at[dsr], sem),
                 pltpu.async_copy(cy_hbm.at[crow3_i.at[r]],
                                  cx3y_v.at[dsr], sem),
                 pltpu.async_copy(cz_hbm.at[crow3_i.at[r]],
                                  cx3z_v.at[dsr], sem)]
            for cc in g:
                cc.wait()
            return 0

        def st(r, _):
            for k in range(8):
                o = r * 128 + k * L
                crow0_i[r, pl.ds(k * L, L)] = crow0_v[pl.ds(o, L)]
                crow3_i[r, pl.ds(k * L, L)] = crow3_v[pl.ds(o, L)]
            return 0
        lax.fori_loop(0, nb, st, 0)
        lax.fori_loop(0, nb, cg, 0)

        def l2(r, _):
            for k in range(8):
                o = r * 128 + k * L
                dx = cx0x_v[pl.ds(o, L)] - cx3x_v[pl.ds(o, L)]
                dy = cx0y_v[pl.ds(o, L)] - cx3y_v[pl.ds(o, L)]
                dz = cx0z_v[pl.ds(o, L)] - cx3z_v[pl.ds(o, L)]
                d2 = dx * dx + dy * dy + dz * dz + 1e-12
                bits = lax.bitcast_convert_type(d2, jnp.int32)
                yb = jnp.int32(0x5F3759DF) - lax.shift_right_arithmetic(bits, 1)
                y = lax.bitcast_convert_type(yb, jnp.float32)
                y = y * (1.5 - 0.5 * d2 * y * y)
                y = y * (1.5 - 0.5 * d2 * y * y)
                y = y * (1.5 - 0.5 * d2 * y * y)
                d = d2 * y
                pp0 = cp0_v[pl.ds(o, L)]
                pp1 = cp1_v[pl.ds(o, L)]
                pp2 = cp2_v[pl.ds(o, L)]
                fnv = lax.shift_right_logical(cib_v[pl.ds(o, L)], 10) & 1
                t = (d - 5.0 * pp0) / (pp1 + 0.5)
                s0 = t * t
                lb = 2.0 * pp0
                ub = lb + 4.0 * pp2 + 1.0
                e1 = jnp.maximum(lb - d, 0.0)
                e2 = jnp.maximum(d - ub, 0.0)
                s1 = e1 * e1 + e2 * e2
                csval_v[pl.ds(o, L)] = jnp.where(fnv == 0, s0, s1)
            return 0
        lax.fori_loop(0, nb, l2, 0)

        for j in range(8):
            csval_v[pl.ds(cnt + j * L, L)] = jnp.zeros((L,), jnp.float32)

        def st2(r, _):
            for k in range(8):
                o = r * 128 + k * L
                csi1_i[r, pl.ds(k * L, L)] = csi1_v[pl.ds(o, L)]
                csi2_i[r, pl.ds(k * L, L)] = csi2_v[pl.ds(o, L)]
                csval_i[r, pl.ds(k * L, L)] = csval_v[pl.ds(o, L)]
            return 0
        lax.fori_loop(0, nb, st2, 0)

        def sca(r, _):
            pltpu.sync_copy(csval_i.at[r], acc_sh.at[csi1_i.at[r]], add=True)
            pltpu.sync_copy(csval_i.at[r], acc_sh.at[csi2_i.at[r]], add=True)
            return 0
        for p in range(NPER):
            plsc.subcore_barrier()
            @pl.when(parity == p)
            def _():
                lax.fori_loop(0, nb, sca, 0)
        return 0

    lax.fori_loop(0, NCHUNK, chunk, 0)

    plsc.subcore_barrier()

    obase = sub * SL
    hbase = core * ACC + obase
    nblk = SL // 2048
    tail = SL - nblk * 2048

    def red_block(off, size):
        def zc(i, _):
            obuf_v[pl.ds(i * L, L)] = jnp.zeros((L,), jnp.float32)
            return 0
        lax.fori_loop(0, size // L, zc, 0)
        def rg(g, _):
            pltpu.sync_copy(
                acc_sh.at[pl.ds(g * ACC + obase + off, size)],
                tbuf_v.at[pl.ds(0, size)])
            def av(i, _):
                obuf_v[pl.ds(i * L, L)] = (obuf_v[pl.ds(i * L, L)]
                                           + tbuf_v[pl.ds(i * L, L)])
                return 0
            lax.fori_loop(0, size // L, av, 0)
            return 0
        lax.fori_loop(0, NG, rg, 0)
        pltpu.sync_copy(obuf_v.at[pl.ds(0, size)],
                        out_hbm.at[pl.ds(hbase + off, size)])

    def wo(i, _):
        red_block(i * 2048, 2048)
        return 0
    lax.fori_loop(0, nblk, wo, 0)
    if tail:
        red_block(nblk * 2048, tail)


def _add_body(a_ref, o_ref):
    o_ref[...] = a_ref[0] + a_ref[1]


def kernel(coords, constraint_params, block_coord_offset, constraint_atoms,
           constraint_function_inds, block_pair_dispatch_indices):
    assert coords.shape == (NP, MA, 3)
    assert constraint_atoms.shape == (NC, 4, 3)
    assert block_pair_dispatch_indices.shape == (3, ND)
    assert block_coord_offset.shape == (NP, NB)

    cf = coords.reshape(NP * MA, 3)
    cx = cf[:, 0]
    cy = cf[:, 1]
    cz = cf[:, 2]
    bco = block_coord_offset.reshape(-1).astype(jnp.int32)

    pose0 = constraint_atoms[:, 0, 0]
    pose3 = constraint_atoms[:, 3, 0]
    r0 = constraint_atoms[:, 0, 1]
    a0 = constraint_atoms[:, 0, 2]
    r3 = constraint_atoms[:, 3, 1]
    a3 = constraint_atoms[:, 3, 2]
    fni = constraint_function_inds
    ia = r3 + (r0 << 11) + (pose0 << 22) + (pose3 << 26)
    ib = a0 + (a3 << 5) + (fni << 10)
    padc = NCP - NC
    pads = lambda x: jnp.pad(x, (0, padc))
    ia = pads(ia)
    ib = pads(ib)
    p0 = pads(constraint_params[:, 0])
    p1 = pads(constraint_params[:, 1])
    p2 = pads(constraint_params[:, 2])

    d0 = block_pair_dispatch_indices[0]
    d1 = block_pair_dispatch_indices[1]
    d2 = block_pair_dispatch_indices[2]
    dflat = d0 * NBB + d1 * NB + d2
    dflat_pad = jnp.pad(dflat, (0, DISP_PAD - ND))

    lookup = pl.kernel(
        _phase1_body,
        out_type=jax.ShapeDtypeStruct((TABLE,), jnp.int32),
        mesh=_mesh(),
        compiler_params=_SC_PARAMS,
        scratch_types=[
            pltpu.VMEM((DPW,), jnp.int32),
            pltpu.VMEM((DPW // 128, 128), jnp.int32),
            pltpu.VMEM((DPW // 128, 128), jnp.int32),
            pltpu.SemaphoreType.DMA,
        ],
    )(dflat_pad)

    partials = pl.kernel(
        _phase2_body,
        out_type=jax.ShapeDtypeStruct((NCORES * ACC,), jnp.float32),
        mesh=_mesh(),
        compiler_params=_SC_PARAMS,
        scratch_types=[
            pltpu.VMEM((NP * NB,), jnp.int32),
            pltpu.VMEM((CH,), jnp.int32),
            pltpu.VMEM((CH,), jnp.int32),
            pltpu.VMEM((CH,), jnp.float32),
            pltpu.VMEM((CH,), jnp.float32),
            pltpu.VMEM((CH,), jnp.float32),
            pltpu.VMEM((CH,), jnp.int32),
            pltpu.VMEM((CH,), jnp.int32),
            pltpu.VMEM((CH,), jnp.int32),
            pltpu.VMEM((CH,), jnp.int32),
            pltpu.VMEM((CH,), jnp.int32),
            pltpu.VMEM((CH,), jnp.int32),
            pltpu.VMEM((CCAP,), jnp.int32),
            pltpu.VMEM((CCAP,), jnp.int32),
            pltpu.VMEM((CCAP,), jnp.int32),
            pltpu.VMEM((CCAP,), jnp.int32),
            pltpu.VMEM((CCAP,), jnp.float32),
            pltpu.VMEM((CCAP,), jnp.float32),
            pltpu.VMEM((CCAP,), jnp.float32),
            pltpu.VMEM((CCAP,), jnp.int32),
            pltpu.VMEM((CCAP,), jnp.float32),
            pltpu.VMEM((CCAP,), jnp.float32),
            pltpu.VMEM((CCAP,), jnp.float32),
            pltpu.VMEM((CCAP,), jnp.float32),
            pltpu.VMEM((CCAP,), jnp.float32),
            pltpu.VMEM((CCAP,), jnp.float32),
            pltpu.VMEM((CCAP,), jnp.float32),
            pltpu.VMEM((CROWS, 128), jnp.int32),
            pltpu.VMEM((CROWS, 128), jnp.int32),
            pltpu.VMEM((CROWS, 128), jnp.int32),
            pltpu.VMEM((CROWS, 128), jnp.int32),
            pltpu.VMEM((CROWS, 128), jnp.float32),
            pltpu.VMEM((2048,), jnp.float32),
            pltpu.VMEM((2048,), jnp.float32),
            pltpu.VMEM_SHARED((NG * ACC,), jnp.float32),
            pltpu.SemaphoreType.DMA,
            pltpu.SemaphoreType.DMA,
        ],
    )(cx, cy, cz, bco, ia, ib, p0, p1, p2, lookup, dflat_pad)

    summed = pl.pallas_call(
        _add_body,
        out_shape=jax.ShapeDtypeStruct((ACC // 128, 128), jnp.float32),
    )(partials.reshape(NCORES, ACC // 128, 128))

    return summed.reshape(-1)[:ND]
